# baseline (speedup 1.0000x reference)
import functools

import jax
import jax.numpy as jnp
from jax import lax
from jax.experimental import pallas as pl
from jax.experimental.pallas import tpu as pltpu
from jax.experimental.pallas import tpu_sc as plsc

H = 64
S = 200
LANES = 16
NC = 2
NS = 16
NW = NC * NS
CHUNK = 128
NBUF = 4


def _body(ids_hbm, table_hbm, pos_hbm, out_hbm,
          idx_v, rows_v, pos2_v, sem_i, sem_g, sem_o):
    wid = lax.axis_index("s") * NC + lax.axis_index("c")
    rows_per_w = ids_hbm.shape[0] // NW
    nchunk = rows_per_w // CHUNK
    ngroup = nchunk // NBUF
    base = wid * rows_per_w

    pltpu.sync_copy(pos_hbm, pos2_v.at[pl.ds(0, S)])
    pltpu.sync_copy(pos_hbm, pos2_v.at[pl.ds(S, S)])

    def idx_copy(c, b):
        return pltpu.make_async_copy(
            ids_hbm.at[pl.ds(base + c * CHUNK, CHUNK)], idx_v.at[b],
            sem_i.at[b])

    def out_copy(c, b):
        return pltpu.make_async_copy(
            rows_v.at[b], out_hbm.at[pl.ds(base + c * CHUNK, CHUNK)],
            sem_o.at[b])

    for b in range(NBUF):
        idx_copy(b, b).start()

    def group(g, _):
        c0 = g * NBUF
        for b in range(NBUF):

            @pl.when(g > 0)
            def _():
                out_copy(c0 + b, b).wait()

            idx_copy(c0 + b, b).wait()
            pltpu.make_async_copy(
                table_hbm.at[idx_v.at[b]], rows_v.at[b], sem_g.at[b]).start()

        for b in range(NBUF):
            c = c0 + b
            pltpu.make_async_copy(
                table_hbm.at[idx_v.at[b]], rows_v.at[b], sem_g.at[b]).wait()

            off = lax.rem(c * CHUNK, S)

            def add_row(i, _):
                for j in range(H // LANES):
                    sl = pl.ds(j * LANES, LANES)
                    plsc.addupdate(rows_v.at[b, i, sl], pos2_v[off + i, sl])
                return 0

            lax.fori_loop(0, CHUNK, add_row, 0)
            out_copy(c, b).start()

            @pl.when(g < ngroup - 1)
            def _():
                idx_copy(c + NBUF, b).start()

        return 0

    lax.fori_loop(0, ngroup, group, 0)


@jax.jit
def _sc_embed(ids_flat, item_table, pos_table):
    n = ids_flat.shape[0]
    mesh = plsc.VectorSubcoreMesh(core_axis_name="c", subcore_axis_name="s")
    kern = functools.partial(
        pl.kernel,
        out_type=jax.ShapeDtypeStruct((n, H), jnp.float32),
        mesh=mesh,
        scratch_types=[
            pltpu.VMEM((NBUF, CHUNK), jnp.int32),
            pltpu.VMEM((NBUF, CHUNK, H), jnp.float32),
            pltpu.VMEM((2 * S, H), jnp.float32),
            pltpu.SemaphoreType.DMA((NBUF,)),
            pltpu.SemaphoreType.DMA((NBUF,)),
            pltpu.SemaphoreType.DMA((NBUF,)),
        ],
        compiler_params=pltpu.CompilerParams(use_tc_tiling_on_sc=False),
    )(_body)
    return kern(ids_flat, item_table, pos_table)


def kernel(input_ids, item_table, pos_table):
    bsz, seq = input_ids.shape
    ids_flat = input_ids.reshape(-1).astype(jnp.int32)
    out = _sc_embed(ids_flat, item_table, pos_table)
    return out.reshape(bsz, seq, H)
